# baseline (speedup 1.0000x reference)
import dataclasses
import functools

import jax
import jax.numpy as jnp
from jax import lax
from jax.experimental import pallas as pl
from jax.experimental.pallas import tpu as pltpu
from jax.experimental.pallas import tpu_sc as plsc

D = 128
DP = D // 2
NC = 2
NS = 16
NW = NC * NS
CHUNK = 80


@jax.jit
def _link_dot(xu_packed, xi_packed, idx_u, idx_v):
    E = idx_u.shape[0]
    epw = E // NW
    nchunks = epw // CHUNK
    groups = CHUNK // 16

    mesh = plsc.VectorSubcoreMesh(core_axis_name="c", subcore_axis_name="s")
    cp = pltpu.CompilerParams()
    if "needs_layout_passes" in pltpu.CompilerParams.__dataclass_fields__:
        cp = dataclasses.replace(cp, needs_layout_passes=False)
    if "use_tc_tiling_on_sc" in pltpu.CompilerParams.__dataclass_fields__:
        cp = dataclasses.replace(cp, use_tc_tiling_on_sc=False)
    if "disable_bounds_checks" in pltpu.CompilerParams.__dataclass_fields__:
        cp = dataclasses.replace(cp, disable_bounds_checks=True)

    @functools.partial(
        pl.kernel,
        out_type=jax.ShapeDtypeStruct((E,), jnp.float32),
        mesh=mesh,
        compiler_params=cp,
        scratch_types=[
            pltpu.VMEM((epw,), jnp.int32),
            pltpu.VMEM((epw,), jnp.int32),
            pltpu.VMEM((2, CHUNK, DP), jnp.int32),
            pltpu.VMEM((2, CHUNK, DP), jnp.int32),
            pltpu.VMEM((epw,), jnp.float32),
            pltpu.SemaphoreType.DMA,
            pltpu.SemaphoreType.DMA,
            pltpu.SemaphoreType.DMA,
            pltpu.SemaphoreType.DMA,
        ],
    )
    def k(xu_hbm, xi_hbm, iu_hbm, iv_hbm, out_hbm,
          iu_all, iv_all, u_bufs, v_bufs, out_all,
          su0, sv0, su1, sv1):
        wid = lax.axis_index("s") * NC + lax.axis_index("c")
        base = wid * epw
        sem_u = (su0, su1)
        sem_v = (sv0, sv1)

        pltpu.sync_copy(iu_hbm.at[pl.ds(base, epw)], iu_all)
        pltpu.sync_copy(iv_hbm.at[pl.ds(base, epw)], iv_all)

        def issue(g, b):
            pltpu.async_copy(
                xu_hbm.at[iu_all.at[pl.ds(g * CHUNK, CHUNK)]],
                u_bufs.at[b], sem_u[b])
            pltpu.async_copy(
                xi_hbm.at[iv_all.at[pl.ds(g * CHUNK, CHUNK)]],
                v_bufs.at[b], sem_v[b])

        def wait(g, b):
            pltpu.make_async_copy(
                xu_hbm.at[iu_all.at[pl.ds(g * CHUNK, CHUNK)]],
                u_bufs.at[b], sem_u[b]).wait()
            pltpu.make_async_copy(
                xi_hbm.at[iv_all.at[pl.ds(g * CHUNK, CHUNK)]],
                v_bufs.at[b], sem_v[b]).wait()

        def compute(g, b):
            ub = u_bufs.at[b]
            vb = v_bufs.at[b]

            @pl.loop(0, groups)
            def _group(t):
                rows = t * 16 + lax.iota(jnp.int32, 16)

                def dim_body(_, carry):
                    acc_lo, acc_hi, cols = carry
                    pu = plsc.load_gather(ub, [rows, cols])
                    pv = plsc.load_gather(vb, [rows, cols])
                    ul = plsc.bitcast(pu << 16, jnp.float32)
                    vl = plsc.bitcast(pv << 16, jnp.float32)
                    uh = plsc.bitcast(pu, jnp.float32)
                    vh = plsc.bitcast(pv, jnp.float32)
                    return (acc_lo + ul * vl, acc_hi + uh * vh,
                            (cols + 1) & (DP - 1))

                zero = jnp.zeros((16,), jnp.float32)
                acc_lo, acc_hi, _ = lax.fori_loop(
                    0, DP, dim_body,
                    (zero, zero, lax.iota(jnp.int32, 16)),
                    unroll=8)
                out_all[pl.ds(g * CHUNK + t * 16, 16)] = acc_lo + acc_hi

        issue(0, 0)

        @pl.loop(0, (nchunks - 1) // 2)
        def _pair(i):
            g = 2 * i
            issue(g + 1, 1)
            wait(g, 0)
            compute(g, 0)
            issue(g + 2, 0)
            wait(g + 1, 1)
            compute(g + 1, 1)

        wait(nchunks - 1, 0)
        compute(nchunks - 1, 0)

        pltpu.sync_copy(out_all, out_hbm.at[pl.ds(base, epw)])

    return k(xu_packed, xi_packed, idx_u, idx_v)


def _pack_bf16(x):
    half = x.shape[1] // 2
    lo = lax.bitcast_convert_type(
        x[:, :half].astype(jnp.bfloat16), jnp.uint16).astype(jnp.uint32)
    hi = lax.bitcast_convert_type(
        x[:, half:].astype(jnp.bfloat16), jnp.uint16).astype(jnp.uint32)
    return lax.bitcast_convert_type(lo | (hi << 16), jnp.int32)


def kernel(x_user, x_item, edge_label_index):
    idx = edge_label_index.astype(jnp.int32)
    return _link_dot(_pack_bf16(x_user), _pack_bf16(x_item), idx[0], idx[1])


# speedup vs baseline: 1.0482x; 1.0482x over previous
import dataclasses
import functools

import jax
import jax.numpy as jnp
from jax import lax
from jax.experimental import pallas as pl
from jax.experimental.pallas import tpu as pltpu
from jax.experimental.pallas import tpu_sc as plsc

D = 128
DP = D // 2
NC = 2
NS = 16
NW = NC * NS
CHUNK = 80


@jax.jit
def _link_dot(xu_packed, xi_packed, idx_u, idx_v):
    E = idx_u.shape[0]
    epw = E // NW
    nchunks = epw // CHUNK
    groups = CHUNK // 16

    mesh = plsc.VectorSubcoreMesh(core_axis_name="c", subcore_axis_name="s")
    cp = pltpu.CompilerParams()
    if "needs_layout_passes" in pltpu.CompilerParams.__dataclass_fields__:
        cp = dataclasses.replace(cp, needs_layout_passes=False)
    if "use_tc_tiling_on_sc" in pltpu.CompilerParams.__dataclass_fields__:
        cp = dataclasses.replace(cp, use_tc_tiling_on_sc=False)
    if "disable_bounds_checks" in pltpu.CompilerParams.__dataclass_fields__:
        cp = dataclasses.replace(cp, disable_bounds_checks=True)

    @functools.partial(
        pl.kernel,
        out_type=jax.ShapeDtypeStruct((E,), jnp.float32),
        mesh=mesh,
        compiler_params=cp,
        scratch_types=[
            pltpu.VMEM((epw,), jnp.int32),
            pltpu.VMEM((epw,), jnp.int32),
            pltpu.VMEM((2, CHUNK, DP), jnp.int32),
            pltpu.VMEM((2, CHUNK, DP), jnp.int32),
            pltpu.VMEM((epw,), jnp.float32),
            pltpu.SemaphoreType.DMA,
            pltpu.SemaphoreType.DMA,
            pltpu.SemaphoreType.DMA,
            pltpu.SemaphoreType.DMA,
        ],
    )
    def k(xu_hbm, xi_hbm, iu_hbm, iv_hbm, out_hbm,
          iu_all, iv_all, u_bufs, v_bufs, out_all,
          su0, sv0, su1, sv1):
        wid = lax.axis_index("s") * NC + lax.axis_index("c")
        base = wid * epw
        sem_u = (su0, su1)
        sem_v = (sv0, sv1)

        pltpu.sync_copy(iu_hbm.at[pl.ds(base, epw)], iu_all)
        pltpu.sync_copy(iv_hbm.at[pl.ds(base, epw)], iv_all)

        def issue(g, b):
            pltpu.async_copy(
                xu_hbm.at[iu_all.at[pl.ds(g * CHUNK, CHUNK)]],
                u_bufs.at[b], sem_u[b])
            pltpu.async_copy(
                xi_hbm.at[iv_all.at[pl.ds(g * CHUNK, CHUNK)]],
                v_bufs.at[b], sem_v[b])

        def wait(g, b):
            pltpu.make_async_copy(
                xu_hbm.at[iu_all.at[pl.ds(g * CHUNK, CHUNK)]],
                u_bufs.at[b], sem_u[b]).wait()
            pltpu.make_async_copy(
                xi_hbm.at[iv_all.at[pl.ds(g * CHUNK, CHUNK)]],
                v_bufs.at[b], sem_v[b]).wait()

        def compute(g, b):
            ub = u_bufs.at[b]
            vb = v_bufs.at[b]
            lane = lax.iota(jnp.int32, 16)
            rows = tuple(t * 16 + lane for t in range(groups))
            zero = jnp.zeros((16,), jnp.float32)

            def col_body(_, carry):
                accs, cols = carry
                new_accs = []
                for t in range(groups):
                    acc_lo, acc_hi = accs[t]
                    pu = plsc.load_gather(ub, [rows[t], cols])
                    pv = plsc.load_gather(vb, [rows[t], cols])
                    ul = plsc.bitcast(pu << 16, jnp.float32)
                    vl = plsc.bitcast(pv << 16, jnp.float32)
                    uh = plsc.bitcast(pu, jnp.float32)
                    vh = plsc.bitcast(pv, jnp.float32)
                    new_accs.append((acc_lo + ul * vl, acc_hi + uh * vh))
                return tuple(new_accs), (cols + 1) & (DP - 1)

            accs, _ = lax.fori_loop(
                0, DP, col_body,
                (tuple((zero, zero) for _ in range(groups)), lane),
                unroll=4)
            for t in range(groups):
                acc_lo, acc_hi = accs[t]
                out_all[pl.ds(g * CHUNK + t * 16, 16)] = acc_lo + acc_hi

        issue(0, 0)

        @pl.loop(0, (nchunks - 1) // 2)
        def _pair(i):
            g = 2 * i
            issue(g + 1, 1)
            wait(g, 0)
            compute(g, 0)
            issue(g + 2, 0)
            wait(g + 1, 1)
            compute(g + 1, 1)

        wait(nchunks - 1, 0)
        compute(nchunks - 1, 0)

        pltpu.sync_copy(out_all, out_hbm.at[pl.ds(base, epw)])

    return k(xu_packed, xi_packed, idx_u, idx_v)


def _pack_bf16(x):
    half = x.shape[1] // 2
    lo = lax.bitcast_convert_type(
        x[:, :half].astype(jnp.bfloat16), jnp.uint16).astype(jnp.uint32)
    hi = lax.bitcast_convert_type(
        x[:, half:].astype(jnp.bfloat16), jnp.uint16).astype(jnp.uint32)
    return lax.bitcast_convert_type(lo | (hi << 16), jnp.int32)


def kernel(x_user, x_item, edge_label_index):
    idx = edge_label_index.astype(jnp.int32)
    return _link_dot(_pack_bf16(x_user), _pack_bf16(x_item), idx[0], idx[1])


# speedup vs baseline: 1.2553x; 1.1975x over previous
import dataclasses
import functools

import jax
import jax.numpy as jnp
from jax import lax
from jax.experimental import pallas as pl
from jax.experimental.pallas import tpu as pltpu
from jax.experimental.pallas import tpu_sc as plsc

D = 128
DP = D // 2
NC = 2
NS = 16
NW = NC * NS
CHUNK = 80


@jax.jit
def _link_dot(xu_packed, xi_packed, idx_u, idx_v):
    E = idx_u.shape[0]
    epw = E // NW
    nchunks = epw // CHUNK
    groups = CHUNK // 16

    mesh = plsc.VectorSubcoreMesh(core_axis_name="c", subcore_axis_name="s")
    cp = pltpu.CompilerParams()
    if "needs_layout_passes" in pltpu.CompilerParams.__dataclass_fields__:
        cp = dataclasses.replace(cp, needs_layout_passes=False)
    if "use_tc_tiling_on_sc" in pltpu.CompilerParams.__dataclass_fields__:
        cp = dataclasses.replace(cp, use_tc_tiling_on_sc=False)
    if "disable_bounds_checks" in pltpu.CompilerParams.__dataclass_fields__:
        cp = dataclasses.replace(cp, disable_bounds_checks=True)

    @functools.partial(
        pl.kernel,
        out_type=jax.ShapeDtypeStruct((E,), jnp.float32),
        mesh=mesh,
        compiler_params=cp,
        scratch_types=[
            pltpu.VMEM((epw,), jnp.int32),
            pltpu.VMEM((epw,), jnp.int32),
            pltpu.VMEM((4, CHUNK, DP), jnp.int32),
            pltpu.VMEM((4, CHUNK, DP), jnp.int32),
            pltpu.VMEM((epw,), jnp.float32),
            pltpu.SemaphoreType.DMA,
            pltpu.SemaphoreType.DMA,
            pltpu.SemaphoreType.DMA,
            pltpu.SemaphoreType.DMA,
            pltpu.SemaphoreType.DMA,
            pltpu.SemaphoreType.DMA,
            pltpu.SemaphoreType.DMA,
            pltpu.SemaphoreType.DMA,
        ],
    )
    def k(xu_hbm, xi_hbm, iu_hbm, iv_hbm, out_hbm,
          iu_all, iv_all, u_bufs, v_bufs, out_all,
          su0, sv0, su1, sv1, su2, sv2, su3, sv3):
        wid = lax.axis_index("s") * NC + lax.axis_index("c")
        base = wid * epw
        sem_u = (su0, su1, su2, su3)
        sem_v = (sv0, sv1, sv2, sv3)

        pltpu.sync_copy(iu_hbm.at[pl.ds(base, epw)], iu_all)
        pltpu.sync_copy(iv_hbm.at[pl.ds(base, epw)], iv_all)

        def issue(g, b):
            pltpu.async_copy(
                xu_hbm.at[iu_all.at[pl.ds(g * CHUNK, CHUNK)]],
                u_bufs.at[b], sem_u[b])
            pltpu.async_copy(
                xi_hbm.at[iv_all.at[pl.ds(g * CHUNK, CHUNK)]],
                v_bufs.at[b], sem_v[b])

        def wait(g, b):
            pltpu.make_async_copy(
                xu_hbm.at[iu_all.at[pl.ds(g * CHUNK, CHUNK)]],
                u_bufs.at[b], sem_u[b]).wait()
            pltpu.make_async_copy(
                xi_hbm.at[iv_all.at[pl.ds(g * CHUNK, CHUNK)]],
                v_bufs.at[b], sem_v[b]).wait()

        def compute(g, b):
            ub = u_bufs.at[b]
            vb = v_bufs.at[b]
            lane = lax.iota(jnp.int32, 16)
            rows = tuple(t * 16 + lane for t in range(groups))
            zero = jnp.zeros((16,), jnp.float32)

            def col_body(_, carry):
                accs, cols = carry
                new_accs = []
                for t in range(groups):
                    acc_lo, acc_hi = accs[t]
                    pu = plsc.load_gather(ub, [rows[t], cols])
                    pv = plsc.load_gather(vb, [rows[t], cols])
                    ul = plsc.bitcast(pu << 16, jnp.float32)
                    vl = plsc.bitcast(pv << 16, jnp.float32)
                    uh = plsc.bitcast(pu, jnp.float32)
                    vh = plsc.bitcast(pv, jnp.float32)
                    new_accs.append((acc_lo + ul * vl, acc_hi + uh * vh))
                return tuple(new_accs), (cols + 1) & (DP - 1)

            accs, _ = lax.fori_loop(
                0, DP, col_body,
                (tuple((zero, zero) for _ in range(groups)), lane),
                unroll=4)
            for t in range(groups):
                acc_lo, acc_hi = accs[t]
                out_all[pl.ds(g * CHUNK + t * 16, 16)] = acc_lo + acc_hi

        issue(0, 0)
        issue(1, 1)
        issue(2, 2)

        body = (nchunks - 5) // 4

        @pl.loop(0, body)
        def _ring(i):
            for b in range(4):
                g = 4 * i + b
                wait(g, b)
                compute(g, b)
                issue(g + 3, (b + 3) % 4)

        tail0 = 4 * body
        for j in range(nchunks - tail0):
            g = tail0 + j
            b = g % 4
            wait(g, b)
            compute(g, b)
            if g + 3 < nchunks:
                issue(g + 3, (g + 3) % 4)

        pltpu.sync_copy(out_all, out_hbm.at[pl.ds(base, epw)])

    return k(xu_packed, xi_packed, idx_u, idx_v)


def _pack_bf16(x):
    half = x.shape[1] // 2
    lo = lax.bitcast_convert_type(
        x[:, :half].astype(jnp.bfloat16), jnp.uint16).astype(jnp.uint32)
    hi = lax.bitcast_convert_type(
        x[:, half:].astype(jnp.bfloat16), jnp.uint16).astype(jnp.uint32)
    return lax.bitcast_convert_type(lo | (hi << 16), jnp.int32)


def kernel(x_user, x_item, edge_label_index):
    idx = edge_label_index.astype(jnp.int32)
    return _link_dot(_pack_bf16(x_user), _pack_bf16(x_item), idx[0], idx[1])
